# baseline (speedup 1.0000x reference)
import functools

import jax
import jax.numpy as jnp
from jax import lax
from jax.experimental import pallas as pl
from jax.experimental.pallas import tpu as pltpu
from jax.experimental.pallas import tpu_sc as plsc


def _edge_kernel_call(x_atm, src, dst, edge_attr, W_gate, b_gate,
                      N, E, D, DE):
    info = plsc.get_sparse_core_info()
    NC, NS, L = info.num_cores, info.num_subcores, info.num_lanes
    NW = NC * NS
    CH = 64

    epw = (E + NW - 1) // NW
    epw = (epw + 6 * CH - 1) // (6 * CH) * (6 * CH)
    e_pad = epw * NW
    cpw = epw // CH

    rps = ((N + 1) + NS - 1) // NS
    rps = (rps + 7) // 8 * 8
    npad = rps * NS
    segs = []
    off = 0
    while off < rps:
        segs.append((off, min(CH, rps - off)))
        off += min(CH, rps - off)

    pad_e = e_pad - E
    src_p = jnp.concatenate([src, jnp.zeros((pad_e,), jnp.int32)])
    dst_p = jnp.concatenate([dst, jnp.full((pad_e,), N, jnp.int32)])
    ea_p = jnp.zeros((e_pad, L), edge_attr.dtype)
    ea_p = ea_p.at[:E, :DE].set(edge_attr)

    mesh = plsc.VectorSubcoreMesh(core_axis_name="c", subcore_axis_name="s")

    @functools.partial(
        pl.kernel,
        out_type=jax.ShapeDtypeStruct((NC, npad, D), jnp.float32),
        mesh=mesh,
        scratch_types=(
            [pltpu.VMEM((CH, D), jnp.float32)] * 2
            + [pltpu.VMEM((CH,), jnp.int32)] * 3
            + [pltpu.VMEM((CH,), jnp.int32)] * 3
            + [pltpu.VMEM((CH, 16), jnp.float32)] * 3
            + [
                pltpu.VMEM((DE, D), jnp.float32),
                pltpu.VMEM((D,), jnp.float32),
                pltpu.VMEM_SHARED((npad, D), jnp.float32),
            ]
            + [pltpu.SemaphoreType.DMA] * 10
        ),
    )
    def edge_kernel(x_hbm, src_hbm, dst_hbm, ea_hbm, wg_hbm, bg_hbm,
                    out_hbm, rows0, rows1, srci0, srci1, srci2,
                    dsti0, dsti1, dsti2, eab0, eab1, eab2, wg_v, bg_v,
                    agg_sh, s_ei0, s_ei1, s_ei2, s_ea0, s_ea1, s_ea2,
                    s_g0, s_g1, s_s0, s_s1):
        cid = lax.axis_index("c")
        sid = lax.axis_index("s")
        wid = sid * NC + cid

        pltpu.sync_copy(wg_hbm, wg_v)
        pltpu.sync_copy(bg_hbm, bg_v)

        def zero_rows(r, carry):
            for j in range(D // L):
                rows0[r, pl.ds(j * L, L)] = jnp.zeros((L,), jnp.float32)
            return carry

        lax.fori_loop(0, CH, zero_rows, 0)
        for off, size in segs:
            if size == CH:
                pltpu.sync_copy(rows0, agg_sh.at[pl.ds(sid * rps + off, CH)])
            else:
                pltpu.sync_copy(rows0.at[pl.ds(0, size)],
                                agg_sh.at[pl.ds(sid * rps + off, size)])
        plsc.subcore_barrier()

        ebase = wid * epw
        NJ = D // L
        rows = [rows0, rows1]
        srcb = [srci0, srci1, srci2]
        dstb = [dsti0, dsti1, dsti2]
        eab = [eab0, eab1, eab2]
        s_ei = [s_ei0, s_ei1, s_ei2]
        s_ea = [s_ea0, s_ea1, s_ea2]
        s_g = [s_g0, s_g1]
        s_s = [s_s0, s_s1]

        def compute(rows_v, ea_v):
            @plsc.parallel_loop(0, CH, unroll=2)
            def edge_body(e):
                eav = ea_v[e, :]
                ea = [eav[k] for k in range(DE)]
                rr = [rows_v[e, pl.ds(j * L, L)] for j in range(NJ)]
                us = [ea[0] * wg_v[0, pl.ds(j * L, L)]
                      + ea[1] * wg_v[1, pl.ds(j * L, L)]
                      + ea[2] * wg_v[2, pl.ds(j * L, L)]
                      + ea[3] * wg_v[3, pl.ds(j * L, L)]
                      + bg_v[pl.ds(j * L, L)]
                      for j in range(NJ)]
                gs = [1.0 / (1.0 + jnp.exp(-u)) for u in us]
                for j in range(NJ):
                    rows_v[e, pl.ds(j * L, L)] = rr[j] * gs[j]

        def issue_idx(g, slot):
            base = ebase + g * CH
            pltpu.async_copy(src_hbm.at[pl.ds(base, CH)], srcb[slot],
                             s_ei[slot])
            pltpu.async_copy(dst_hbm.at[pl.ds(base, CH)], dstb[slot],
                             s_ei[slot])
            pltpu.async_copy(ea_hbm.at[pl.ds(base, CH)], eab[slot],
                             s_ea[slot])

        def wait_idx(slot):
            pltpu.make_async_copy(src_hbm.at[pl.ds(0, CH)], srcb[slot],
                                  s_ei[slot]).wait()
            pltpu.make_async_copy(dst_hbm.at[pl.ds(0, CH)], dstb[slot],
                                  s_ei[slot]).wait()

        def stage(g, b2, b3):
            n2 = 1 - b2
            import jax as _jax

            with _jax.named_scope("drain_scatter"):
                @pl.when(g >= 1)
                def _():
                    pltpu.make_async_copy(
                        rows[n2], agg_sh.at[dstb[(b3 - 1) % 3]], s_s[n2]).wait()

            with _jax.named_scope("issue_gather"):
                @pl.when(g + 1 < cpw)
                def _():
                    wait_idx((b3 + 1) % 3)
                    pltpu.async_copy(x_hbm.at[srcb[(b3 + 1) % 3]], rows[n2],
                                     s_g[n2])

            with _jax.named_scope("wait_gather"):
                pltpu.make_async_copy(x_hbm.at[srcb[b3]], rows[b2],
                                      s_g[b2]).wait()

            with _jax.named_scope("prefetch_idx"):
                @pl.when(g + 2 < cpw)
                def _():
                    issue_idx(g + 2, (b3 + 2) % 3)

            with _jax.named_scope("wait_ea"):
                pltpu.make_async_copy(ea_hbm.at[pl.ds(0, CH)], eab[b3],
                                      s_ea[b3]).wait()
            with _jax.named_scope("gate_compute"):
                compute(rows[b2], eab[b3])

            with _jax.named_scope("issue_scatter"):
                pltpu.async_copy(rows[b2], agg_sh.at[dstb[b3]], s_s[b2],
                                 add=True)

        issue_idx(0, 0)
        issue_idx(1, 1)
        wait_idx(0)
        pltpu.async_copy(x_hbm.at[srci0], rows0, s_g0)

        def six_body(q, carry):
            for i in range(6):
                stage(6 * q + i, i % 2, i % 3)
            return carry

        lax.fori_loop(0, cpw // 6, six_body, 0)
        pltpu.make_async_copy(rows1, agg_sh.at[dsti2], s_s1).wait()
        plsc.subcore_barrier()

        for off, size in segs:
            o = sid * rps + off
            pltpu.sync_copy(agg_sh.at[pl.ds(o, size)],
                            out_hbm.at[cid, pl.ds(o, size)])

    return edge_kernel(x_atm, src_p, dst_p, ea_p, W_gate, b_gate), npad


def kernel(x_eq_linear, x_eq, x_atm, edge_index, edge_attr, mask, batch,
           W_gate, b_gate, eps, W1, b1, W2, b2, W_eq, b_eq, W_out, b_out,
           W_lin, b_lin):
    N, D = x_atm.shape
    E = edge_index.shape[1]
    DE = edge_attr.shape[1]
    G, DEQ = x_eq.shape

    src = edge_index[0]
    dst = edge_index[1]

    partials, npad = _edge_kernel_call(
        x_atm, src, dst, edge_attr, W_gate, b_gate, N, E, D, DE)

    batch2 = batch.reshape(1, N)
    mask2 = mask.reshape(1, N)
    eps2 = eps.reshape(1, 1)
    b1_2 = b1.reshape(1, D)
    b2_2 = b2.reshape(1, D)
    beq2 = b_eq.reshape(1, -1)
    bout2 = b_out.reshape(1, 1)
    blin2 = b_lin.reshape(1, 1)

    def tc_body(p_ref, x_ref, batch_ref, mask_ref, eps_ref, w1_ref, b1_ref,
                w2_ref, b2_ref, xeq_ref, weq_ref, beq_ref, wout_ref,
                bout_ref, xlin_ref, wlin_ref, blin_ref, out_ref):
        agg = p_ref[0, :N, :] + p_ref[1, :N, :]
        e = eps_ref[0, 0]
        h = (1.0 + e) * x_ref[...] + agg
        h = jnp.maximum(
            jnp.dot(h, w1_ref[...], preferred_element_type=jnp.float32)
            + b1_ref[...], 0.0)
        h = jnp.maximum(
            jnp.dot(h, w2_ref[...], preferred_element_type=jnp.float32)
            + b2_ref[...], 0.0)
        iota_g = lax.broadcasted_iota(jnp.int32, (G, N), 0)
        onehot = jnp.where(iota_g == batch_ref[...], 1.0, 0.0) * mask_ref[...]
        gemb = jnp.dot(onehot, h, preferred_element_type=jnp.float32)
        eq = jnp.maximum(
            jnp.dot(xeq_ref[...], weq_ref[...],
                    preferred_element_type=jnp.float32) + beq_ref[...], 0.0)
        z = (jnp.dot(gemb, wout_ref[:D, :],
                     preferred_element_type=jnp.float32)
             + jnp.dot(eq, wout_ref[D:, :],
                       preferred_element_type=jnp.float32)
             + bout_ref[...])
        out_ref[...] = (z + jnp.dot(xlin_ref[...], wlin_ref[...],
                                    preferred_element_type=jnp.float32)
                        + blin_ref[...])

    out = pl.pallas_call(
        tc_body,
        out_shape=jax.ShapeDtypeStruct((G, 1), jnp.float32),
    )(partials, x_atm, batch2, mask2, eps2, W1, b1_2, W2, b2_2,
      x_eq, W_eq, beq2, W_out, bout2, x_eq_linear, W_lin, blin2)
    return out


# speedup vs baseline: 1.2657x; 1.2657x over previous
import functools

import jax
import jax.numpy as jnp
from jax import lax
from jax.experimental import pallas as pl
from jax.experimental.pallas import tpu as pltpu
from jax.experimental.pallas import tpu_sc as plsc


def _edge_kernel_call(x_atm, src, dst, edge_attr, W_gate, b_gate,
                      N, E, D, DE):
    info = plsc.get_sparse_core_info()
    NC, NS, L = info.num_cores, info.num_subcores, info.num_lanes
    NW = NC * NS
    CH = 64

    epw = E // NW
    cpw = (epw // CH) // 6 * 6
    tail = epw - cpw * CH
    tails = []
    off = 0
    while off < tail:
        tails.append((cpw * CH + off, min(128, tail - off)))
        off += min(128, tail - off)

    rps = (N + NS - 1) // NS
    rps = (rps + 7) // 8 * 8
    npad = rps * NS
    segs = []
    off = 0
    while off < rps:
        segs.append((off, min(CH, rps - off)))
        off += min(CH, rps - off)

    ea_flat = edge_attr.reshape(-1)

    mesh = plsc.VectorSubcoreMesh(core_axis_name="c", subcore_axis_name="s")

    tail_scratch = []
    for _, tsz in tails:
        tail_scratch += [
            pltpu.VMEM((tsz, D), jnp.float32),
            pltpu.VMEM((tsz,), jnp.int32),
            pltpu.VMEM((tsz,), jnp.int32),
            pltpu.VMEM((tsz * DE,), jnp.float32),
        ]

    @functools.partial(
        pl.kernel,
        out_type=jax.ShapeDtypeStruct((NC, npad, D), jnp.float32),
        mesh=mesh,
        scratch_types=(
            [pltpu.VMEM((CH, D), jnp.float32)] * 2
            + [pltpu.VMEM((CH,), jnp.int32)] * 3
            + [pltpu.VMEM((CH,), jnp.int32)] * 3
            + [pltpu.VMEM((CH * DE,), jnp.float32)] * 3
            + [
                pltpu.VMEM((DE, D), jnp.float32),
                pltpu.VMEM((D,), jnp.float32),
                pltpu.VMEM_SHARED((npad, D), jnp.float32),
            ]
            + tail_scratch
            + [pltpu.SemaphoreType.DMA] * 10
        ),
    )
    def edge_kernel(x_hbm, src_hbm, dst_hbm, ea_hbm, wg_hbm, bg_hbm,
                    out_hbm, *refs):
        rows = list(refs[0:2])
        srcb = list(refs[2:5])
        dstb = list(refs[5:8])
        eab = list(refs[8:11])
        wg_v, bg_v, agg_sh = refs[11:14]
        trefs = refs[14:14 + 4 * len(tails)]
        (s_ei0, s_ei1, s_ei2, s_ea0, s_ea1, s_ea2,
         s_g0, s_g1, s_s0, s_s1) = refs[14 + 4 * len(tails):]
        s_ei = [s_ei0, s_ei1, s_ei2]
        s_ea = [s_ea0, s_ea1, s_ea2]
        s_g = [s_g0, s_g1]
        s_s = [s_s0, s_s1]

        cid = lax.axis_index("c")
        sid = lax.axis_index("s")
        wid = sid * NC + cid

        pltpu.sync_copy(wg_hbm, wg_v)
        pltpu.sync_copy(bg_hbm, bg_v)

        def zero_rows(r, carry):
            for j in range(D // L):
                rows[0][r, pl.ds(j * L, L)] = jnp.zeros((L,), jnp.float32)
            return carry

        lax.fori_loop(0, CH, zero_rows, 0)
        for off, size in segs:
            if size == CH:
                pltpu.sync_copy(rows[0],
                                agg_sh.at[pl.ds(sid * rps + off, CH)])
            else:
                pltpu.sync_copy(rows[0].at[pl.ds(0, size)],
                                agg_sh.at[pl.ds(sid * rps + off, size)])
        plsc.subcore_barrier()

        ebase = wid * epw
        NJ = D // L

        def compute(rows_v, ea_v, n):
            @plsc.parallel_loop(0, n // 4, unroll=1)
            def grp(q):
                eav = ea_v[pl.ds(q * 4 * DE, L)]
                for r in range(4):
                    e = q * 4 + r
                    ea = [eav[4 * r + k] for k in range(DE)]
                    rr = [rows_v[e, pl.ds(j * L, L)] for j in range(NJ)]
                    us = [ea[0] * wg_v[0, pl.ds(j * L, L)]
                          + ea[1] * wg_v[1, pl.ds(j * L, L)]
                          + ea[2] * wg_v[2, pl.ds(j * L, L)]
                          + ea[3] * wg_v[3, pl.ds(j * L, L)]
                          + bg_v[pl.ds(j * L, L)]
                          for j in range(NJ)]
                    gs = [1.0 / (1.0 + jnp.exp(-u)) for u in us]
                    for j in range(NJ):
                        rows_v[e, pl.ds(j * L, L)] = rr[j] * gs[j]

        def issue_idx(g, slot):
            base = ebase + g * CH
            pltpu.async_copy(src_hbm.at[pl.ds(base, CH)], srcb[slot],
                             s_ei[slot])
            pltpu.async_copy(dst_hbm.at[pl.ds(base, CH)], dstb[slot],
                             s_ei[slot])
            pltpu.async_copy(ea_hbm.at[pl.ds(base * DE, CH * DE)],
                             eab[slot], s_ea[slot])

        def wait_idx(slot):
            pltpu.make_async_copy(src_hbm.at[pl.ds(0, CH)], srcb[slot],
                                  s_ei[slot]).wait()
            pltpu.make_async_copy(dst_hbm.at[pl.ds(0, CH)], dstb[slot],
                                  s_ei[slot]).wait()

        def stage(g, b2, b3):
            n2 = 1 - b2

            @pl.when(g >= 1)
            def _():
                pltpu.make_async_copy(
                    rows[n2], agg_sh.at[dstb[(b3 - 1) % 3]], s_s[n2]).wait()

            @pl.when(g + 1 < cpw)
            def _():
                wait_idx((b3 + 1) % 3)
                pltpu.async_copy(x_hbm.at[srcb[(b3 + 1) % 3]], rows[n2],
                                 s_g[n2])

            pltpu.make_async_copy(x_hbm.at[srcb[b3]], rows[b2],
                                  s_g[b2]).wait()

            @pl.when(g + 2 < cpw)
            def _():
                issue_idx(g + 2, (b3 + 2) % 3)

            pltpu.make_async_copy(ea_hbm.at[pl.ds(0, CH * DE)], eab[b3],
                                  s_ea[b3]).wait()
            compute(rows[b2], eab[b3], CH)

            pltpu.async_copy(rows[b2], agg_sh.at[dstb[b3]], s_s[b2],
                             add=True)

        issue_idx(0, 0)
        issue_idx(1, 1)
        wait_idx(0)
        pltpu.async_copy(x_hbm.at[srcb[0]], rows[0], s_g[0])

        def six_body(q, carry):
            for i in range(6):
                stage(6 * q + i, i % 2, i % 3)
            return carry

        lax.fori_loop(0, cpw // 6, six_body, 0)
        pltpu.make_async_copy(rows[1], agg_sh.at[dstb[2]], s_s[1]).wait()

        for t, (toff, tsz) in enumerate(tails):
            rows_t, src_t, dst_t, ea_t = trefs[4 * t:4 * t + 4]
            tbase = ebase + toff
            pltpu.sync_copy(src_hbm.at[pl.ds(tbase, tsz)], src_t)
            pltpu.sync_copy(dst_hbm.at[pl.ds(tbase, tsz)], dst_t)
            pltpu.sync_copy(ea_hbm.at[pl.ds(tbase * DE, tsz * DE)], ea_t)
            pltpu.async_copy(x_hbm.at[src_t], rows_t, s_g[0]).wait()
            compute(rows_t, ea_t, tsz)
            pltpu.sync_copy(rows_t, agg_sh.at[dst_t], add=True)

        plsc.subcore_barrier()

        for off, size in segs:
            o = sid * rps + off
            pltpu.sync_copy(agg_sh.at[pl.ds(o, size)],
                            out_hbm.at[cid, pl.ds(o, size)])

    return edge_kernel(x_atm, src, dst, ea_flat, W_gate, b_gate), npad


def kernel(x_eq_linear, x_eq, x_atm, edge_index, edge_attr, mask, batch,
           W_gate, b_gate, eps, W1, b1, W2, b2, W_eq, b_eq, W_out, b_out,
           W_lin, b_lin):
    N, D = x_atm.shape
    E = edge_index.shape[1]
    DE = edge_attr.shape[1]
    G, DEQ = x_eq.shape

    src = edge_index[0]
    dst = edge_index[1]

    partials, npad = _edge_kernel_call(
        x_atm, src, dst, edge_attr, W_gate, b_gate, N, E, D, DE)

    batch2 = batch.reshape(1, N)
    mask2 = mask.reshape(1, N)
    eps2 = eps.reshape(1, 1)
    b1_2 = b1.reshape(1, D)
    b2_2 = b2.reshape(1, D)
    beq2 = b_eq.reshape(1, -1)
    bout2 = b_out.reshape(1, 1)
    blin2 = b_lin.reshape(1, 1)

    def tc_body(p_ref, x_ref, batch_ref, mask_ref, eps_ref, w1_ref, b1_ref,
                w2_ref, b2_ref, xeq_ref, weq_ref, beq_ref, wout_ref,
                bout_ref, xlin_ref, wlin_ref, blin_ref, out_ref):
        agg = p_ref[0, :N, :] + p_ref[1, :N, :]
        e = eps_ref[0, 0]
        h = (1.0 + e) * x_ref[...] + agg
        h = jnp.maximum(
            jnp.dot(h, w1_ref[...], preferred_element_type=jnp.float32)
            + b1_ref[...], 0.0)
        h = jnp.maximum(
            jnp.dot(h, w2_ref[...], preferred_element_type=jnp.float32)
            + b2_ref[...], 0.0)
        iota_g = lax.broadcasted_iota(jnp.int32, (G, N), 0)
        onehot = jnp.where(iota_g == batch_ref[...], 1.0, 0.0) * mask_ref[...]
        gemb = jnp.dot(onehot, h, preferred_element_type=jnp.float32)
        eq = jnp.maximum(
            jnp.dot(xeq_ref[...], weq_ref[...],
                    preferred_element_type=jnp.float32) + beq_ref[...], 0.0)
        z = (jnp.dot(gemb, wout_ref[:D, :],
                     preferred_element_type=jnp.float32)
             + jnp.dot(eq, wout_ref[D:, :],
                       preferred_element_type=jnp.float32)
             + bout_ref[...])
        out_ref[...] = (z + jnp.dot(xlin_ref[...], wlin_ref[...],
                                    preferred_element_type=jnp.float32)
                        + blin_ref[...])

    out = pl.pallas_call(
        tc_body,
        out_shape=jax.ShapeDtypeStruct((G, 1), jnp.float32),
    )(partials, x_atm, batch2, mask2, eps2, W1, b1_2, W2, b2_2,
      x_eq, W_eq, beq2, W_out, bout2, x_eq_linear, W_lin, blin2)
    return out
